# baseline (speedup 1.0000x reference)
import functools

import jax
import jax.numpy as jnp
from jax import lax
from jax.experimental import pallas as pl
from jax.experimental.pallas import tpu as pltpu
from jax.experimental.pallas import tpu_sc as plsc

_EMB_DIM = 128
_NUM_SEG = 26
_BATCH = 16384
_ROW = _NUM_SEG * _EMB_DIM
_CHUNK_ROWS = 8
_NBUF = 4


def _permute_sc(pooled_embs):
    info = plsc.get_sparse_core_info()
    num_workers = info.num_cores * info.num_subcores
    rows_per_w = _BATCH // num_workers
    n_steps = rows_per_w // _CHUNK_ROWS
    mesh = plsc.VectorSubcoreMesh(core_axis_name="c", subcore_axis_name="s")

    @functools.partial(
        pl.kernel,
        mesh=mesh,
        out_type=jax.ShapeDtypeStruct((_BATCH, _ROW), jnp.float32),
        scratch_types=(
            [pltpu.VMEM((_CHUNK_ROWS, _ROW), jnp.float32)] * _NBUF
            + [pltpu.SemaphoreType.DMA] * (2 * _NBUF)
        ),
    )
    def k(in_hbm, out_hbm, *scr):
        bufs = scr[:_NBUF]
        gsems = scr[_NBUF : 2 * _NBUF]
        ssems = scr[2 * _NBUF :]
        wid = lax.axis_index("s") * info.num_cores + lax.axis_index("c")
        row_base = wid * rows_per_w

        def gathers(t, slot):
            rows = pl.ds(row_base + t * _CHUNK_ROWS, _CHUNK_ROWS)
            for j in range(_NUM_SEG):
                src = (_NUM_SEG - 1 - j) * _EMB_DIM
                pltpu.make_async_copy(
                    in_hbm.at[rows, pl.ds(src, _EMB_DIM)],
                    bufs[slot].at[:, pl.ds(j * _EMB_DIM, _EMB_DIM)],
                    gsems[slot],
                ).start()

        def scatter(t, slot):
            pltpu.make_async_copy(
                bufs[slot],
                out_hbm.at[pl.ds(row_base + t * _CHUNK_ROWS, _CHUNK_ROWS), :],
                ssems[slot],
            ).start()

        dummy_in = in_hbm.at[pl.ds(0, _CHUNK_ROWS), :]
        dummy_out = out_hbm.at[pl.ds(0, _CHUNK_ROWS), :]

        def wait_gathers(slot):
            pltpu.make_async_copy(dummy_in, bufs[slot], gsems[slot]).wait()

        def wait_scatter(slot):
            pltpu.make_async_copy(bufs[slot], dummy_out, ssems[slot]).wait()

        gathers(0, 0)
        gathers(1, 1)
        wait_gathers(0)
        scatter(0, 0)
        gathers(2, 2)
        wait_gathers(1)
        scatter(1, 1)
        gathers(3, 3)

        def body(kk, carry):
            for b in range(4):
                t = 4 * kk + 2 + b
                slot = (2 + b) % 4
                wait_gathers(slot)
                scatter(t, slot)
                wait_scatter(b % 4)
                gathers(t + 2, b % 4)
            return carry

        lax.fori_loop(0, (n_steps - 4) // 4, body, 0)

        wait_gathers(2)
        scatter(n_steps - 2, 2)
        wait_gathers(3)
        scatter(n_steps - 1, 3)
        for slot in range(4):
            wait_scatter(slot)

    return k(pooled_embs)


def kernel(pooled_embs):
    return _permute_sc(pooled_embs)


# speedup vs baseline: 1.0041x; 1.0041x over previous
import functools

import jax
import jax.numpy as jnp
from jax import lax
from jax.experimental import pallas as pl
from jax.experimental.pallas import tpu as pltpu
from jax.experimental.pallas import tpu_sc as plsc

_EMB_DIM = 128
_NUM_SEG = 26
_BATCH = 16384
_ROW = _NUM_SEG * _EMB_DIM
_CHUNK_ROWS = 128
_NBUF = 4


def _permute_sc(pooled_embs):
    info = plsc.get_sparse_core_info()
    num_workers = info.num_cores * info.num_subcores
    rows_per_w = _BATCH // num_workers
    n_rchunks = rows_per_w // _CHUNK_ROWS
    assert n_rchunks == _NBUF
    mesh = plsc.VectorSubcoreMesh(core_axis_name="c", subcore_axis_name="s")

    @functools.partial(
        pl.kernel,
        mesh=mesh,
        out_type=jax.ShapeDtypeStruct((_BATCH, _ROW), jnp.float32),
        scratch_types=(
            [pltpu.VMEM((_CHUNK_ROWS, _EMB_DIM), jnp.float32)] * _NBUF
            + [pltpu.SemaphoreType.DMA] * (2 * _NBUF)
        ),
    )
    def k(in_hbm, out_hbm, *scr):
        bufs = scr[:_NBUF]
        gsems = scr[_NBUF : 2 * _NBUF]
        ssems = scr[2 * _NBUF :]
        wid = lax.axis_index("s") * info.num_cores + lax.axis_index("c")
        row_base = wid * rows_per_w

        def gather(j, c, slot):
            src_col = (_NUM_SEG - 1 - j) * _EMB_DIM
            h = pltpu.make_async_copy(
                in_hbm.at[
                    pl.ds(row_base + c * _CHUNK_ROWS, _CHUNK_ROWS),
                    pl.ds(src_col, _EMB_DIM),
                ],
                bufs[slot],
                gsems[slot],
            )
            h.start()
            return h

        def scatter(j, c, slot):
            h = pltpu.make_async_copy(
                bufs[slot],
                out_hbm.at[
                    pl.ds(row_base + c * _CHUNK_ROWS, _CHUNK_ROWS),
                    pl.ds(j * _EMB_DIM, _EMB_DIM),
                ],
                ssems[slot],
            )
            h.start()
            return h

        dummy_in = in_hbm.at[pl.ds(0, _CHUNK_ROWS), pl.ds(0, _EMB_DIM)]
        dummy_out = out_hbm.at[pl.ds(0, _CHUNK_ROWS), pl.ds(0, _EMB_DIM)]

        def wait_gather(slot):
            pltpu.make_async_copy(dummy_in, bufs[slot], gsems[slot]).wait()

        def wait_scatter(slot):
            pltpu.make_async_copy(bufs[slot], dummy_out, ssems[slot]).wait()

        gather(0, 0, 0)
        gather(0, 1, 1)
        wait_gather(0)
        scatter(0, 0, 0)
        gather(0, 2, 2)
        wait_gather(1)
        scatter(0, 1, 1)
        gather(0, 3, 3)

        def body(kk, carry):
            for b in range(4):
                if b < 2:
                    j, c = kk, 2 + b
                else:
                    j, c = kk + 1, b - 2
                slot = (2 + b) % 4
                wait_gather(slot)
                scatter(j, c, slot)
                wait_scatter(b)
                gather(kk + 1, b, b)
            return carry

        lax.fori_loop(0, _NUM_SEG - 1, body, 0)

        wait_gather(2)
        scatter(_NUM_SEG - 1, 2, 2)
        wait_gather(3)
        scatter(_NUM_SEG - 1, 3, 3)
        for slot in range(4):
            wait_scatter(slot)

    return k(pooled_embs)


def kernel(pooled_embs):
    return _permute_sc(pooled_embs)


# speedup vs baseline: 1.0059x; 1.0018x over previous
import functools

import jax
import jax.numpy as jnp
from jax import lax
from jax.experimental import pallas as pl
from jax.experimental.pallas import tpu as pltpu
from jax.experimental.pallas import tpu_sc as plsc

_EMB_DIM = 128
_NUM_SEG = 26
_BATCH = 16384
_ROW = _NUM_SEG * _EMB_DIM
_CHUNK_ROWS = 128
_NBUF = 4


def _permute_sc(pooled_embs):
    info = plsc.get_sparse_core_info()
    num_workers = info.num_cores * info.num_subcores
    rows_per_w = _BATCH // num_workers
    n_rchunks = rows_per_w // _CHUNK_ROWS
    assert n_rchunks == _NBUF
    mesh = plsc.VectorSubcoreMesh(core_axis_name="c", subcore_axis_name="s")

    @functools.partial(
        pl.kernel,
        mesh=mesh,
        out_type=jax.ShapeDtypeStruct((_BATCH, _ROW), jnp.float32),
        scratch_types=(
            [pltpu.VMEM((_CHUNK_ROWS, _EMB_DIM), jnp.float32)] * _NBUF
            + [pltpu.SemaphoreType.DMA] * (2 * _NBUF)
        ),
    )
    def k(in_hbm, out_hbm, *scr):
        bufs = scr[:_NBUF]
        gsems = scr[_NBUF : 2 * _NBUF]
        ssems = scr[2 * _NBUF :]
        wid = lax.axis_index("s") * info.num_cores + lax.axis_index("c")
        row_base = wid * rows_per_w
        stag = lax.rem(wid, _NUM_SEG)

        def rot(j):
            jr = j + stag
            return jnp.where(jr >= _NUM_SEG, jr - _NUM_SEG, jr)

        def gather(j, c, slot):
            src_col = (_NUM_SEG - 1 - rot(j)) * _EMB_DIM
            h = pltpu.make_async_copy(
                in_hbm.at[
                    pl.ds(row_base + c * _CHUNK_ROWS, _CHUNK_ROWS),
                    pl.ds(src_col, _EMB_DIM),
                ],
                bufs[slot],
                gsems[slot],
            )
            h.start()
            return h

        def scatter(j, c, slot):
            h = pltpu.make_async_copy(
                bufs[slot],
                out_hbm.at[
                    pl.ds(row_base + c * _CHUNK_ROWS, _CHUNK_ROWS),
                    pl.ds(rot(j) * _EMB_DIM, _EMB_DIM),
                ],
                ssems[slot],
            )
            h.start()
            return h

        dummy_in = in_hbm.at[pl.ds(0, _CHUNK_ROWS), pl.ds(0, _EMB_DIM)]
        dummy_out = out_hbm.at[pl.ds(0, _CHUNK_ROWS), pl.ds(0, _EMB_DIM)]

        def wait_gather(slot):
            pltpu.make_async_copy(dummy_in, bufs[slot], gsems[slot]).wait()

        def wait_scatter(slot):
            pltpu.make_async_copy(bufs[slot], dummy_out, ssems[slot]).wait()

        gather(0, 0, 0)
        gather(0, 1, 1)
        wait_gather(0)
        scatter(0, 0, 0)
        gather(0, 2, 2)
        wait_gather(1)
        scatter(0, 1, 1)
        gather(0, 3, 3)

        def body(kk, carry):
            for b in range(4):
                if b < 2:
                    j, c = kk, 2 + b
                else:
                    j, c = kk + 1, b - 2
                slot = (2 + b) % 4
                wait_gather(slot)
                scatter(j, c, slot)
                wait_scatter(b)
                gather(kk + 1, b, b)
            return carry

        lax.fori_loop(0, _NUM_SEG - 1, body, 0)

        wait_gather(2)
        scatter(_NUM_SEG - 1, 2, 2)
        wait_gather(3)
        scatter(_NUM_SEG - 1, 3, 3)
        for slot in range(4):
            wait_scatter(slot)

    return k(pooled_embs)


def kernel(pooled_embs):
    return _permute_sc(pooled_embs)
